# baseline (speedup 1.0000x reference)
import functools
import jax
import jax.numpy as jnp
from jax import lax
from jax.experimental import pallas as pl
from jax.experimental.pallas import tpu as pltpu
from jax.experimental.pallas import tpu_sc as plsc


def _make_sc_kernel(b, h, w, d):
    mesh = plsc.VectorSubcoreMesh(core_axis_name="c", subcore_axis_name="s")
    info = plsc.get_sparse_core_info()
    nc, ns = info.num_cores, info.num_subcores
    rows_per_tile = h // ns

    @functools.partial(
        pl.kernel,
        mesh=mesh,
        out_type=jax.ShapeDtypeStruct((b, h, w, 2 * d), jnp.float32),
        scratch_types=[
            pltpu.VMEM((rows_per_tile, w, 2 * d), jnp.float32),
            pltpu.VMEM_SHARED((h, w, 2 * d), jnp.float32),
            pltpu.SemaphoreType.DMA,
        ],
    )
    def k(col_hbm, row_hbm, out_hbm, tile_v, shared, sem):
        cid = lax.axis_index("c")
        sid = lax.axis_index("s")
        y0 = sid * rows_per_tile
        fills = []
        for rr in range(rows_per_tile):
            for xx in range(w):
                fills.append(pltpu.make_async_copy(
                    col_hbm.at[xx], tile_v.at[rr, xx, pl.ds(0, d)], sem))
                fills.append(pltpu.make_async_copy(
                    row_hbm.at[y0 + rr], tile_v.at[rr, xx, pl.ds(d, d)], sem))
        for f in fills:
            f.start()
        for f in fills:
            f.wait()
        pltpu.sync_copy(tile_v, shared.at[pl.ds(y0, rows_per_tile)])
        plsc.subcore_barrier()
        bb = sid * nc + cid
        cp = pltpu.make_async_copy(shared, out_hbm.at[bb], sem)
        cp.start()
        cp.wait()

    return k


def kernel(x, row_emb, col_emb):
    b = x.shape[0]
    h, w = x.shape[-2], x.shape[-1]
    d = row_emb.shape[1]
    k = _make_sc_kernel(b, h, w, d)
    out = k(col_emb, row_emb)
    return jnp.transpose(out, (0, 3, 1, 2))


# speedup vs baseline: 2.9894x; 2.9894x over previous
import jax
import jax.numpy as jnp
from jax.experimental import pallas as pl
from jax.experimental.pallas import tpu as pltpu

_BPG = 2


def kernel(x, row_emb, col_emb):
    b = x.shape[0]
    h, w = x.shape[-2], x.shape[-1]
    d = row_emb.shape[1]

    def body(col_ref, row_ref, out_ref):
        col = col_ref[:w, :]
        row = row_ref[:h, :]
        out_ref[:, :, :, 0:d] = jnp.broadcast_to(
            col[None, None, :, :], (_BPG, h, w, d))
        out_ref[:, :, :, d:2 * d] = jnp.broadcast_to(
            row[None, :, None, :], (_BPG, h, w, d))

    out = pl.pallas_call(
        body,
        grid=(b // _BPG,),
        in_specs=[
            pl.BlockSpec(col_emb.shape, lambda i: (0, 0)),
            pl.BlockSpec(row_emb.shape, lambda i: (0, 0)),
        ],
        out_specs=pl.BlockSpec((_BPG, h, w, 2 * d), lambda i: (i, 0, 0, 0)),
        out_shape=jax.ShapeDtypeStruct((b, h, w, 2 * d), jnp.float32),
    )(col_emb, row_emb)
    return jnp.transpose(out, (0, 3, 1, 2))
